# Initial kernel scaffold; baseline (speedup 1.0000x reference)
import jax
import jax.numpy as jnp
from jax.experimental import pallas as pl


def kernel(query, W_qkv, b_qkv, W_fc, b_fc):
    raise NotImplementedError("write your pallas kernel here")


# speedup vs baseline: 4.6297x; 4.6297x over previous
import jax
import jax.numpy as jnp
from jax.experimental import pallas as pl

_N = 2048
_C = 2048
_H = 16
_D = 128
_U = 40
_SCALE = _D ** -0.5


def _qkv_kernel(x_ref, w_ref, b_ref, out_ref):
    acc = jax.lax.dot_general(
        x_ref[...], w_ref[...], (((1,), (0,)), ((), ())),
        preferred_element_type=jnp.float32)
    acc = acc + b_ref[0].astype(jnp.float32)
    for j in range(4):
        out_ref[0, j, :, :] = acc[:, j * _D:(j + 1) * _D]


def _qkv_proj(x_bf, w_bf, b):
    return pl.pallas_call(
        _qkv_kernel,
        grid=(3, 4),
        in_specs=[
            pl.BlockSpec((_N, _C), lambda s, hh: (0, 0)),
            pl.BlockSpec((_C, 512), lambda s, hh: (0, s * 4 + hh)),
            pl.BlockSpec((1, 1, 512), lambda s, hh: (s * 4 + hh, 0, 0)),
        ],
        out_specs=pl.BlockSpec((1, 4, _N, _D), lambda s, hh: (s, hh, 0, 0)),
        out_shape=jax.ShapeDtypeStruct((3, _H, _N, _D), jnp.float32),
    )(x_bf, w_bf, b)


def _topk_kernel(q_ref, idx_ref):
    q = q_ref[0]
    norms2 = jnp.sum(q * q, axis=-1)
    iota_n = jax.lax.broadcasted_iota(jnp.int32, (_H, _N), 1)
    vals = norms2
    picks = []
    for _ in range(_U):
        m = jnp.max(vals, axis=1, keepdims=True)
        cand = jnp.where(vals == m, iota_n, _N)
        sel = jnp.min(cand, axis=1, keepdims=True)
        picks.append(sel)
        vals = jnp.where(iota_n == sel, -jnp.inf, vals)
    idx = jnp.concatenate(picks, axis=1)
    head = jax.lax.broadcasted_iota(jnp.int32, (_H, _U), 0)
    idx_ref[:, 0, :] = idx + head * _N


def _topk(qkv):
    return pl.pallas_call(
        _topk_kernel,
        grid=(1,),
        in_specs=[pl.BlockSpec((1, _H, _N, _D), lambda i: (0, 0, 0, 0))],
        out_specs=pl.BlockSpec((_H, 1, _U), lambda i: (0, 0, 0)),
        out_shape=jax.ShapeDtypeStruct((_H, 1, _U), jnp.int32),
    )(qkv)


def _attn_kernel(q_ref, k_ref, v_ref, idx_ref, out_ref):
    h = pl.program_id(0)
    q = q_ref[0, 0]
    k = k_ref[0, 0]
    v = v_ref[0, 0]
    local_idx = idx_ref[0, 0:1, :] - h * _N
    iota_n = jax.lax.broadcasted_iota(jnp.int32, (_N, _U), 0)
    onehot_t = (iota_n == local_idx).astype(jnp.float32)

    q_red = jax.lax.dot_general(
        onehot_t, q, (((0,), (0,)), ((), ())), preferred_element_type=jnp.float32)
    s = jax.lax.dot_general(
        q_red, k, (((1,), (1,)), ((), ())),
        preferred_element_type=jnp.float32) * _SCALE

    p = s - jnp.max(s, axis=1, keepdims=True)
    p = jnp.exp(p)
    p = p / jnp.sum(p, axis=1, keepdims=True)
    p2 = p - jnp.max(p, axis=1, keepdims=True)
    p2 = jnp.exp(p2)
    p2 = p2 / jnp.sum(p2, axis=1, keepdims=True)

    rows = jax.lax.dot_general(
        p2, v, (((1,), (0,)), ((), ())), preferred_element_type=jnp.float32)
    scat = jax.lax.dot_general(
        onehot_t, rows, (((1,), (0,)), ((), ())), preferred_element_type=jnp.float32)
    selected = jnp.sum(onehot_t, axis=1, keepdims=True) > 0.0
    out = jnp.where(selected, scat, jnp.nan)
    out_ref[...] = out.astype(jnp.bfloat16)


def _sparse_attn(qkv, idx_flat):
    return pl.pallas_call(
        _attn_kernel,
        grid=(_H,),
        in_specs=[
            pl.BlockSpec((1, 1, _N, _D), lambda h: (0, h, 0, 0)),
            pl.BlockSpec((1, 1, _N, _D), lambda h: (1, h, 0, 0)),
            pl.BlockSpec((1, 1, _N, _D), lambda h: (2, h, 0, 0)),
            pl.BlockSpec((1, 1, _U), lambda h: (h, 0, 0)),
        ],
        out_specs=pl.BlockSpec((_N, _D), lambda h: (0, h)),
        out_shape=jax.ShapeDtypeStruct((_N, _C), jnp.bfloat16),
    )(qkv, qkv, qkv, idx_flat)


def _fc_kernel(x_ref, w_ref, b_ref, out_ref):
    acc = jax.lax.dot_general(
        x_ref[...], w_ref[...], (((1,), (0,)), ((), ())),
        preferred_element_type=jnp.float32)
    out_ref[...] = acc + b_ref[...]


def _fc(attn_out, w_bf, b):
    return pl.pallas_call(
        _fc_kernel,
        grid=(4,),
        in_specs=[
            pl.BlockSpec((512, _C), lambda i: (i, 0)),
            pl.BlockSpec((_C, _C), lambda i: (0, 0)),
            pl.BlockSpec((1, _C), lambda i: (0, 0)),
        ],
        out_specs=pl.BlockSpec((512, _C), lambda i: (i, 0)),
        out_shape=jax.ShapeDtypeStruct((_N, _C), jnp.float32),
    )(attn_out, w_bf, b)


@jax.jit
def kernel(query, W_qkv, b_qkv, W_fc, b_fc):
    B, N, C = query.shape
    x_bf = query.reshape(N, C).astype(jnp.bfloat16)
    qkv = _qkv_proj(x_bf, W_qkv.astype(jnp.bfloat16), b_qkv.reshape(12, 1, 512))
    idx_flat = _topk(qkv)
    attn_out = _sparse_attn(qkv, idx_flat)
    out = _fc(attn_out, W_fc.astype(jnp.bfloat16), b_fc.reshape(1, C))
    return out.reshape(B, N, C)
